# baseline (speedup 1.0000x reference)
import functools

import jax
import jax.numpy as jnp
from jax import lax
from jax.experimental import pallas as pl
from jax.experimental.pallas import tpu as pltpu
from jax.experimental.pallas import tpu_sc as plsc

N = 10000
E = 320000
NHID = 128
NLAYERS = 3
NCLASS = 128
NGRAPHS = 128
NFEATCOLS = 9
ATOM_VOCAB = 120
SCALAR = 0.5
BN_EPS = 1e-5

NC = 2
NS = 16
NW = NC * NS

A_CH = 80
A_NCH = 4
ROWS_PER_TILE = A_CH * A_NCH
NP = ROWS_PER_TILE * NW

E_CH = 128
E_G = 40
E_NG = 2
E_NCH = E_G * E_NG
EP = NW * E_NCH * E_CH

_mesh = plsc.VectorSubcoreMesh(core_axis_name="c", subcore_axis_name="s")


@functools.partial(
    pl.kernel,
    out_type=jax.ShapeDtypeStruct((NP, NHID), jnp.float32),
    mesh=_mesh,
    scratch_types=[
        pltpu.VMEM((NFEATCOLS * A_NCH, A_CH), jnp.int32),
        pltpu.VMEM((ROWS_PER_TILE, NHID), jnp.float32),
        pltpu.SemaphoreType.DMA,
    ],
)
def _atom_encode_sc(hoff_hbm, tables_hbm, x_out, idx_v, acc_v, sem):
    cid = lax.axis_index("c")
    sid = lax.axis_index("s")
    wid = sid * NC + cid
    base = wid * ROWS_PER_TILE
    pltpu.sync_copy(hoff_hbm.at[wid], idx_v)
    for f in range(NFEATCOLS):
        for c in range(A_NCH):
            pltpu.async_copy(
                tables_hbm.at[idx_v.at[f * A_NCH + c]],
                acc_v.at[pl.ds(c * A_CH, A_CH)],
                sem,
                add=(f > 0),
            )
        for c in range(A_NCH):
            pltpu.make_async_copy(
                tables_hbm.at[idx_v.at[f * A_NCH + c]],
                acc_v.at[pl.ds(c * A_CH, A_CH)],
                sem,
            ).wait()
    pltpu.sync_copy(acc_v, x_out.at[pl.ds(base, ROWS_PER_TILE)])


@functools.partial(
    pl.kernel,
    out_type=jax.ShapeDtypeStruct((2 * NP, NHID), jnp.float32),
    mesh=_mesh,
    scratch_types=[
        pltpu.VMEM((E_G, E_CH), jnp.int32),
        pltpu.VMEM((E_G, E_CH), jnp.int32),
        pltpu.VMEM((E_CH, NHID), jnp.float32),
        pltpu.VMEM((E_CH, NHID), jnp.float32),
        pltpu.VMEM_SHARED((NP, NHID), jnp.float32),
        pltpu.SemaphoreType.DMA,
        pltpu.SemaphoreType.DMA,
    ],
)
def _edge_aggregate_sc(x_hbm, src_hbm, dst_hbm, zeros_hbm, p_out,
                       sidx, didx, rows0, rows1, aggr_sh, gsem0, gsem1):
    cid = lax.axis_index("c")
    sid = lax.axis_index("s")
    wid = sid * NC + cid
    rows_per_tile = NP // NS
    pltpu.sync_copy(zeros_hbm.at[pl.ds(sid * rows_per_tile, rows_per_tile)],
                    aggr_sh.at[pl.ds(sid * rows_per_tile, rows_per_tile)])
    plsc.subcore_barrier()

    def stage(c, rows, gsem, issue_next):
        pltpu.make_async_copy(x_hbm.at[sidx.at[c]], rows, gsem).wait()
        pltpu.sync_copy(rows, aggr_sh.at[didx.at[c]], add=True)
        if issue_next:
            pltpu.async_copy(x_hbm.at[sidx.at[c + 2]], rows, gsem)

    def body(i, carry):
        c = 2 * i
        stage(c, rows0, gsem0, True)
        stage(c + 1, rows1, gsem1, True)
        return carry

    for g in range(E_NG):
        pltpu.sync_copy(src_hbm.at[wid * E_NG + g], sidx)
        pltpu.sync_copy(dst_hbm.at[wid * E_NG + g], didx)
        pltpu.async_copy(x_hbm.at[sidx.at[0]], rows0, gsem0)
        pltpu.async_copy(x_hbm.at[sidx.at[1]], rows1, gsem1)
        lax.fori_loop(0, (E_G - 2) // 2, body, 0)
        stage(E_G - 2, rows0, gsem0, False)
        stage(E_G - 1, rows1, gsem1, False)
    plsc.subcore_barrier()
    pltpu.sync_copy(aggr_sh.at[pl.ds(sid * rows_per_tile, rows_per_tile)],
                    p_out.at[pl.ds(cid * NP + sid * rows_per_tile, rows_per_tile)])


def _layer_tc_body(x_ref, p_ref, w_ref, b_ref, g_ref, be_ref, o_ref, *, relu):
    x = x_ref[0:N, :]
    y = (1.0 + SCALAR) * x + p_ref[0:N, :] + p_ref[NP:NP + N, :]
    z = jnp.dot(y, w_ref[:], preferred_element_type=jnp.float32) + b_ref[:]
    mean = jnp.mean(z, axis=0, keepdims=True)
    zc = z - mean
    var = jnp.mean(zc * zc, axis=0, keepdims=True)
    zn = zc * lax.rsqrt(var + BN_EPS) * g_ref[:] + be_ref[:]
    if relu:
        zn = jnp.maximum(zn, 0.0)
    o_ref[0:N, :] = zn


def _layer_tc(x, p, w, b, g, be, relu):
    return pl.pallas_call(
        functools.partial(_layer_tc_body, relu=relu),
        out_shape=jax.ShapeDtypeStruct((NP, NHID), jnp.float32),
    )(x, p, w, b, g, be)


def _pool_tc_body(x_ref, batch_ref, pw_ref, pb_ref, o_ref):
    b = batch_ref[:]
    gids = lax.broadcasted_iota(jnp.int32, (NGRAPHS, N), 0)
    onehot = (gids == b).astype(jnp.float32)
    sums = jnp.dot(onehot, x_ref[0:N, :], preferred_element_type=jnp.float32)
    counts = jnp.maximum(jnp.sum(onehot, axis=1, keepdims=True), 1.0)
    pooled = sums / counts
    o_ref[:] = jnp.dot(pooled, pw_ref[:],
                       preferred_element_type=jnp.float32) + pb_ref[:]


def _pool_tc(x, batch2, pw, pb):
    return pl.pallas_call(
        _pool_tc_body,
        out_shape=jax.ShapeDtypeStruct((NGRAPHS, NCLASS), jnp.float32),
    )(x, batch2, pw, pb)


def kernel(h, edge_index, pair_info, batch, atom_tables, conv_W, conv_b,
           bn_gamma, bn_beta, pred_W, pred_b):
    hp = jnp.pad(h, ((0, NP - N), (0, 0)))
    hoff = hp + (jnp.arange(NFEATCOLS, dtype=jnp.int32) * ATOM_VOCAB)[None, :]
    hoff = (hoff.T.reshape(NFEATCOLS, NW, A_NCH, A_CH)
            .transpose(1, 0, 2, 3).reshape(NW, NFEATCOLS * A_NCH, A_CH))
    tables_flat = atom_tables.reshape(NFEATCOLS * ATOM_VOCAB, NHID)
    pad_dst = N + jnp.arange(EP - E, dtype=jnp.int32) % (NP - N)
    src = jnp.concatenate(
        [pair_info[0], jnp.zeros((EP - E,), jnp.int32)]
    ).reshape(NW * E_NG, E_G, E_CH)
    dst = jnp.concatenate([pair_info[1], pad_dst]).reshape(NW * E_NG, E_G, E_CH)
    zeros = jnp.zeros((NP, NHID), jnp.float32)
    batch2 = batch.reshape(1, N)

    x = _atom_encode_sc(hoff, tables_flat)
    for layer in range(NLAYERS):
        p = _edge_aggregate_sc(x, src, dst, zeros)
        x = _layer_tc(x, p, conv_W[layer], conv_b[layer].reshape(1, NHID),
                      bn_gamma[layer].reshape(1, NHID),
                      bn_beta[layer].reshape(1, NHID),
                      relu=layer < NLAYERS - 1)
    return _pool_tc(x, batch2, pred_W, pred_b.reshape(1, NCLASS))


# speedup vs baseline: 2.9314x; 2.9314x over previous
import functools

import jax
import jax.numpy as jnp
from jax import lax
from jax.experimental import pallas as pl
from jax.experimental.pallas import tpu as pltpu
from jax.experimental.pallas import tpu_sc as plsc

N = 10000
E = 320000
NHID = 128
NLAYERS = 3
NCLASS = 128
NGRAPHS = 128
NFEATCOLS = 9
ATOM_VOCAB = 120
SCALAR = 0.5
BN_EPS = 1e-5

NC = 2
NS = 16
NW = NC * NS

A_CH = 80
A_NCH = 4
ROWS_PER_TILE = A_CH * A_NCH
NP = ROWS_PER_TILE * NW

E_CH = 128
E_G = 40
E_NG = 2
E_NCH = E_G * E_NG
EP = NW * E_NCH * E_CH

_mesh = plsc.VectorSubcoreMesh(core_axis_name="c", subcore_axis_name="s")


@functools.partial(
    pl.kernel,
    out_type=jax.ShapeDtypeStruct((NP, NHID), jnp.float32),
    mesh=_mesh,
    scratch_types=[
        pltpu.VMEM((NFEATCOLS * A_NCH, A_CH), jnp.int32),
        pltpu.VMEM((ROWS_PER_TILE, NHID), jnp.float32),
        pltpu.SemaphoreType.DMA,
    ],
)
def _atom_encode_sc(hoff_hbm, tables_hbm, x_out, idx_v, acc_v, sem):
    cid = lax.axis_index("c")
    sid = lax.axis_index("s")
    wid = sid * NC + cid
    base = wid * ROWS_PER_TILE
    pltpu.sync_copy(hoff_hbm.at[wid], idx_v)
    for f in range(NFEATCOLS):
        for c in range(A_NCH):
            pltpu.async_copy(
                tables_hbm.at[idx_v.at[f * A_NCH + c]],
                acc_v.at[pl.ds(c * A_CH, A_CH)],
                sem,
                add=(f > 0),
            )
        for c in range(A_NCH):
            pltpu.make_async_copy(
                tables_hbm.at[idx_v.at[f * A_NCH + c]],
                acc_v.at[pl.ds(c * A_CH, A_CH)],
                sem,
            ).wait()
    pltpu.sync_copy(acc_v, x_out.at[pl.ds(base, ROWS_PER_TILE)])


@functools.partial(
    pl.kernel,
    out_type=jax.ShapeDtypeStruct((2 * NP, NHID), jnp.float32),
    mesh=_mesh,
    scratch_types=[
        pltpu.VMEM((E_G, E_CH), jnp.int32),
        pltpu.VMEM((E_G, E_CH), jnp.int32),
        pltpu.VMEM((E_CH, NHID), jnp.float32),
        pltpu.VMEM((E_CH, NHID), jnp.float32),
        pltpu.VMEM_SHARED((NP, NHID), jnp.float32),
        pltpu.SemaphoreType.DMA,
        pltpu.SemaphoreType.DMA,
    ],
)
def _edge_aggregate_sc(x_hbm, src_hbm, dst_hbm, zeros_hbm, p_out,
                       sidx, didx, rows0, rows1, aggr_sh, gsem0, gsem1):
    cid = lax.axis_index("c")
    sid = lax.axis_index("s")
    wid = sid * NC + cid
    rows_per_tile = NP // NS
    pltpu.sync_copy(zeros_hbm.at[pl.ds(sid * rows_per_tile, rows_per_tile)],
                    aggr_sh.at[pl.ds(sid * rows_per_tile, rows_per_tile)])
    plsc.subcore_barrier()

    def stage(c, rows, gsem, issue_next):
        pltpu.make_async_copy(x_hbm.at[sidx.at[c]], rows, gsem).wait()
        pltpu.sync_copy(rows, aggr_sh.at[didx.at[c]], add=True)
        if issue_next:
            pltpu.async_copy(x_hbm.at[sidx.at[c + 2]], rows, gsem)

    def body(i, carry):
        c = 2 * i
        stage(c, rows0, gsem0, True)
        stage(c + 1, rows1, gsem1, True)
        return carry

    for g in range(E_NG):
        pltpu.sync_copy(src_hbm.at[wid * E_NG + g], sidx)
        pltpu.sync_copy(dst_hbm.at[wid * E_NG + g], didx)
        pltpu.async_copy(x_hbm.at[sidx.at[0]], rows0, gsem0)
        pltpu.async_copy(x_hbm.at[sidx.at[1]], rows1, gsem1)
        lax.fori_loop(0, (E_G - 2) // 2, body, 0)
        stage(E_G - 2, rows0, gsem0, False)
        stage(E_G - 1, rows1, gsem1, False)
    plsc.subcore_barrier()
    pltpu.sync_copy(aggr_sh.at[pl.ds(sid * rows_per_tile, rows_per_tile)],
                    p_out.at[pl.ds(cid * NP + sid * rows_per_tile, rows_per_tile)])


def _layer_tc_body(x_ref, p_ref, w_ref, b_ref, g_ref, be_ref, o_ref, *, relu):
    x = x_ref[0:N, :]
    y = (1.0 + SCALAR) * x + p_ref[0:N, :] + p_ref[NP:NP + N, :]
    z = jnp.dot(y, w_ref[:], preferred_element_type=jnp.float32) + b_ref[:]
    mean = jnp.mean(z, axis=0, keepdims=True)
    zc = z - mean
    var = jnp.mean(zc * zc, axis=0, keepdims=True)
    zn = zc * lax.rsqrt(var + BN_EPS) * g_ref[:] + be_ref[:]
    if relu:
        zn = jnp.maximum(zn, 0.0)
    o_ref[0:N, :] = zn


def _layer_tc(x, p, w, b, g, be, relu):
    return pl.pallas_call(
        functools.partial(_layer_tc_body, relu=relu),
        out_shape=jax.ShapeDtypeStruct((NP, NHID), jnp.float32),
    )(x, p, w, b, g, be)


def _pool_tc_body(x_ref, batch_ref, pw_ref, pb_ref, o_ref):
    b = batch_ref[:]
    gids = lax.broadcasted_iota(jnp.int32, (NGRAPHS, N), 0)
    onehot = (gids == b).astype(jnp.float32)
    sums = jnp.dot(onehot, x_ref[0:N, :], preferred_element_type=jnp.float32)
    counts = jnp.maximum(jnp.sum(onehot, axis=1, keepdims=True), 1.0)
    pooled = sums / counts
    o_ref[:] = jnp.dot(pooled, pw_ref[:],
                       preferred_element_type=jnp.float32) + pb_ref[:]


def _pool_tc(x, batch2, pw, pb):
    return pl.pallas_call(
        _pool_tc_body,
        out_shape=jax.ShapeDtypeStruct((NGRAPHS, NCLASS), jnp.float32),
    )(x, batch2, pw, pb)


def kernel(h, edge_index, pair_info, batch, atom_tables, conv_W, conv_b,
           bn_gamma, bn_beta, pred_W, pred_b):
    hp = jnp.pad(h, ((0, NP - N), (0, 0)))
    hoff = hp + (jnp.arange(NFEATCOLS, dtype=jnp.int32) * ATOM_VOCAB)[None, :]
    hoff = (hoff.T.reshape(NFEATCOLS, NW, A_NCH, A_CH)
            .transpose(1, 0, 2, 3).reshape(NW, NFEATCOLS * A_NCH, A_CH))
    tables_flat = atom_tables.reshape(NFEATCOLS * ATOM_VOCAB, NHID)
    pad_t = NP - N
    pad_src = jnp.broadcast_to(
        jnp.arange(pad_t, dtype=jnp.int32) * (N // pad_t), (NW, pad_t))
    pad_dst = jnp.broadcast_to(
        N + jnp.arange(pad_t, dtype=jnp.int32), (NW, pad_t))
    src = jnp.concatenate(
        [pair_info[0].reshape(NW, E // NW), pad_src], axis=1
    ).reshape(NW * E_NG, E_G, E_CH)
    dst = jnp.concatenate(
        [pair_info[1].reshape(NW, E // NW), pad_dst], axis=1
    ).reshape(NW * E_NG, E_G, E_CH)
    zeros = jnp.zeros((NP, NHID), jnp.float32)
    batch2 = batch.reshape(1, N)

    x = _atom_encode_sc(hoff, tables_flat)
    for layer in range(NLAYERS):
        p = _edge_aggregate_sc(x, src, dst, zeros)
        x = _layer_tc(x, p, conv_W[layer], conv_b[layer].reshape(1, NHID),
                      bn_gamma[layer].reshape(1, NHID),
                      bn_beta[layer].reshape(1, NHID),
                      relu=layer < NLAYERS - 1)
    return _pool_tc(x, batch2, pred_W, pred_b.reshape(1, NCLASS))


# speedup vs baseline: 3.8619x; 1.3174x over previous
import functools

import jax
import jax.numpy as jnp
from jax import lax
from jax.experimental import pallas as pl
from jax.experimental.pallas import tpu as pltpu
from jax.experimental.pallas import tpu_sc as plsc

N = 10000
E = 320000
NHID = 128
NLAYERS = 3
NCLASS = 128
NGRAPHS = 128
NFEATCOLS = 9
ATOM_VOCAB = 120
SCALAR = 0.5
BN_EPS = 1e-5

NC = 2
NS = 16
NW = NC * NS

A_CH = 80
A_NCH = 4
ROWS_PER_TILE = A_CH * A_NCH
NP = ROWS_PER_TILE * NW
TBL_PAD = 1152

E_CH = 128
E_G = 40
E_NG = 2
E_NCH = E_G * E_NG
EP = NW * E_NCH * E_CH

_mesh = plsc.VectorSubcoreMesh(core_axis_name="c", subcore_axis_name="s")


@functools.partial(
    pl.kernel,
    out_type=jax.ShapeDtypeStruct((NP, NHID), jnp.float32),
    mesh=_mesh,
    scratch_types=[
        pltpu.VMEM((NFEATCOLS * A_NCH, A_CH), jnp.int32),
        pltpu.VMEM((ROWS_PER_TILE, NHID), jnp.float32),
        pltpu.VMEM_SHARED((TBL_PAD, NHID), jnp.float32),
        pltpu.SemaphoreType.DMA,
    ],
)
def _atom_encode_sc(hoff_hbm, tables_hbm, x_out, idx_v, acc_v, tbl_sh, sem):
    cid = lax.axis_index("c")
    sid = lax.axis_index("s")
    wid = sid * NC + cid
    base = wid * ROWS_PER_TILE
    tpt = TBL_PAD // NS
    pltpu.sync_copy(tables_hbm.at[pl.ds(sid * tpt, tpt)],
                    tbl_sh.at[pl.ds(sid * tpt, tpt)])
    pltpu.sync_copy(hoff_hbm.at[wid], idx_v)
    plsc.subcore_barrier()
    for f in range(NFEATCOLS):
        for c in range(A_NCH):
            pltpu.async_copy(
                tbl_sh.at[idx_v.at[f * A_NCH + c]],
                acc_v.at[pl.ds(c * A_CH, A_CH)],
                sem,
                add=(f > 0),
            )
        for c in range(A_NCH):
            pltpu.make_async_copy(
                tbl_sh.at[idx_v.at[f * A_NCH + c]],
                acc_v.at[pl.ds(c * A_CH, A_CH)],
                sem,
            ).wait()
    pltpu.sync_copy(acc_v, x_out.at[pl.ds(base, ROWS_PER_TILE)])


@functools.partial(
    pl.kernel,
    out_type=jax.ShapeDtypeStruct((2 * NP, NHID), jnp.float32),
    mesh=_mesh,
    scratch_types=[
        pltpu.VMEM((E_G, E_CH), jnp.int32),
        pltpu.VMEM((E_G, E_CH), jnp.int32),
        pltpu.VMEM((E_CH, NHID), jnp.float32),
        pltpu.VMEM((E_CH, NHID), jnp.float32),
        pltpu.VMEM_SHARED((NP, NHID), jnp.float32),
        pltpu.SemaphoreType.DMA,
        pltpu.SemaphoreType.DMA,
    ],
)
def _edge_aggregate_sc(x_hbm, src_hbm, dst_hbm, zeros_hbm, p_out,
                       sidx, didx, rows0, rows1, aggr_sh, gsem0, gsem1):
    cid = lax.axis_index("c")
    sid = lax.axis_index("s")
    wid = sid * NC + cid
    rows_per_tile = NP // NS
    pltpu.sync_copy(zeros_hbm.at[pl.ds(sid * rows_per_tile, rows_per_tile)],
                    aggr_sh.at[pl.ds(sid * rows_per_tile, rows_per_tile)])
    plsc.subcore_barrier()

    def stage(c, rows, gsem, issue_next):
        pltpu.make_async_copy(x_hbm.at[sidx.at[c]], rows, gsem).wait()
        pltpu.sync_copy(rows, aggr_sh.at[didx.at[c]], add=True)
        if issue_next:
            pltpu.async_copy(x_hbm.at[sidx.at[c + 2]], rows, gsem)

    def body(i, carry):
        c = 2 * i
        stage(c, rows0, gsem0, True)
        stage(c + 1, rows1, gsem1, True)
        return carry

    for g in range(E_NG):
        pltpu.sync_copy(src_hbm.at[wid * E_NG + g], sidx)
        pltpu.sync_copy(dst_hbm.at[wid * E_NG + g], didx)
        pltpu.async_copy(x_hbm.at[sidx.at[0]], rows0, gsem0)
        pltpu.async_copy(x_hbm.at[sidx.at[1]], rows1, gsem1)
        lax.fori_loop(0, (E_G - 2) // 2, body, 0)
        stage(E_G - 2, rows0, gsem0, False)
        stage(E_G - 1, rows1, gsem1, False)
    plsc.subcore_barrier()
    pltpu.sync_copy(aggr_sh.at[pl.ds(sid * rows_per_tile, rows_per_tile)],
                    p_out.at[pl.ds(cid * NP + sid * rows_per_tile, rows_per_tile)])


def _layer_tc_body(x_ref, p_ref, w_ref, b_ref, g_ref, be_ref, o_ref, *, relu):
    x = x_ref[0:N, :]
    y = (1.0 + SCALAR) * x + p_ref[0:N, :] + p_ref[NP:NP + N, :]
    z = jnp.dot(y, w_ref[:], preferred_element_type=jnp.float32) + b_ref[:]
    mean = jnp.mean(z, axis=0, keepdims=True)
    zc = z - mean
    var = jnp.mean(zc * zc, axis=0, keepdims=True)
    zn = zc * lax.rsqrt(var + BN_EPS) * g_ref[:] + be_ref[:]
    if relu:
        zn = jnp.maximum(zn, 0.0)
    o_ref[0:N, :] = zn


def _layer_tc(x, p, w, b, g, be, relu):
    return pl.pallas_call(
        functools.partial(_layer_tc_body, relu=relu),
        out_shape=jax.ShapeDtypeStruct((NP, NHID), jnp.float32),
    )(x, p, w, b, g, be)


def _pool_tc_body(x_ref, batch_ref, pw_ref, pb_ref, o_ref):
    b = batch_ref[:]
    gids = lax.broadcasted_iota(jnp.int32, (NGRAPHS, N), 0)
    onehot = (gids == b).astype(jnp.float32)
    sums = jnp.dot(onehot, x_ref[0:N, :], preferred_element_type=jnp.float32)
    counts = jnp.maximum(jnp.sum(onehot, axis=1, keepdims=True), 1.0)
    pooled = sums / counts
    o_ref[:] = jnp.dot(pooled, pw_ref[:],
                       preferred_element_type=jnp.float32) + pb_ref[:]


def _pool_tc(x, batch2, pw, pb):
    return pl.pallas_call(
        _pool_tc_body,
        out_shape=jax.ShapeDtypeStruct((NGRAPHS, NCLASS), jnp.float32),
    )(x, batch2, pw, pb)


def kernel(h, edge_index, pair_info, batch, atom_tables, conv_W, conv_b,
           bn_gamma, bn_beta, pred_W, pred_b):
    hp = jnp.pad(h, ((0, NP - N), (0, 0)))
    hoff = hp + (jnp.arange(NFEATCOLS, dtype=jnp.int32) * ATOM_VOCAB)[None, :]
    hoff = (hoff.T.reshape(NFEATCOLS, NW, A_NCH, A_CH)
            .transpose(1, 0, 2, 3).reshape(NW, NFEATCOLS * A_NCH, A_CH))
    tables_flat = jnp.pad(
        atom_tables.reshape(NFEATCOLS * ATOM_VOCAB, NHID),
        ((0, TBL_PAD - NFEATCOLS * ATOM_VOCAB), (0, 0)))
    pad_t = NP - N
    pad_src = jnp.broadcast_to(
        jnp.arange(pad_t, dtype=jnp.int32) * (N // pad_t), (NW, pad_t))
    pad_dst = jnp.broadcast_to(
        N + jnp.arange(pad_t, dtype=jnp.int32), (NW, pad_t))
    src = jnp.concatenate(
        [pair_info[0].reshape(NW, E // NW), pad_src], axis=1
    ).reshape(NW * E_NG, E_G, E_CH)
    dst = jnp.concatenate(
        [pair_info[1].reshape(NW, E // NW), pad_dst], axis=1
    ).reshape(NW * E_NG, E_G, E_CH)
    zeros = jnp.zeros((NP, NHID), jnp.float32)
    batch2 = batch.reshape(1, N)

    x = _atom_encode_sc(hoff, tables_flat)
    for layer in range(NLAYERS):
        p = _edge_aggregate_sc(x, src, dst, zeros)
        x = _layer_tc(x, p, conv_W[layer], conv_b[layer].reshape(1, NHID),
                      bn_gamma[layer].reshape(1, NHID),
                      bn_beta[layer].reshape(1, NHID),
                      relu=layer < NLAYERS - 1)
    return _pool_tc(x, batch2, pred_W, pred_b.reshape(1, NCLASS))


# speedup vs baseline: 3.9103x; 1.0125x over previous
import functools

import jax
import jax.numpy as jnp
from jax import lax
from jax.experimental import pallas as pl
from jax.experimental.pallas import tpu as pltpu
from jax.experimental.pallas import tpu_sc as plsc

N = 10000
E = 320000
NHID = 128
NLAYERS = 3
NCLASS = 128
NGRAPHS = 128
NFEATCOLS = 9
ATOM_VOCAB = 120
SCALAR = 0.5
BN_EPS = 1e-5

NC = 2
NS = 16
NW = NC * NS

A_CH = 80
A_NCH = 4
ROWS_PER_TILE = A_CH * A_NCH
NP = ROWS_PER_TILE * NW
TBL_PAD = 1152

E_CH = 128
E_G = 40
E_NG = 2
E_NCH = E_G * E_NG
EP = NW * E_NCH * E_CH

_mesh = plsc.VectorSubcoreMesh(core_axis_name="c", subcore_axis_name="s")


@functools.partial(
    pl.kernel,
    out_type=jax.ShapeDtypeStruct((NP, NHID), jnp.float32),
    mesh=_mesh,
    scratch_types=[
        pltpu.VMEM((NFEATCOLS * A_NCH, A_CH), jnp.int32),
        pltpu.VMEM((ROWS_PER_TILE, NHID), jnp.float32),
        pltpu.VMEM_SHARED((TBL_PAD, NHID), jnp.float32),
        pltpu.SemaphoreType.DMA,
    ],
)
def _atom_encode_sc(hoff_hbm, tables_hbm, x_out, idx_v, acc_v, tbl_sh, sem):
    cid = lax.axis_index("c")
    sid = lax.axis_index("s")
    wid = sid * NC + cid
    base = wid * ROWS_PER_TILE
    tpt = TBL_PAD // NS
    pltpu.sync_copy(tables_hbm.at[pl.ds(sid * tpt, tpt)],
                    tbl_sh.at[pl.ds(sid * tpt, tpt)])
    pltpu.sync_copy(hoff_hbm.at[wid], idx_v)
    plsc.subcore_barrier()
    for f in range(NFEATCOLS):
        for c in range(A_NCH):
            pltpu.async_copy(
                tbl_sh.at[idx_v.at[f * A_NCH + c]],
                acc_v.at[pl.ds(c * A_CH, A_CH)],
                sem,
                add=(f > 0),
            )
        for c in range(A_NCH):
            pltpu.make_async_copy(
                tbl_sh.at[idx_v.at[f * A_NCH + c]],
                acc_v.at[pl.ds(c * A_CH, A_CH)],
                sem,
            ).wait()
    pltpu.sync_copy(acc_v, x_out.at[pl.ds(base, ROWS_PER_TILE)])


@functools.partial(
    pl.kernel,
    out_type=jax.ShapeDtypeStruct((2 * NP, NHID), jnp.float32),
    mesh=_mesh,
    scratch_types=[
        pltpu.VMEM((E_G, E_CH), jnp.int32),
        pltpu.VMEM((E_G, E_CH), jnp.int32),
        pltpu.VMEM((E_CH, NHID), jnp.float32),
        pltpu.VMEM((E_CH, NHID), jnp.float32),
        pltpu.VMEM_SHARED((NP, NHID), jnp.float32),
        pltpu.SemaphoreType.DMA,
        pltpu.SemaphoreType.DMA,
    ],
)
def _edge_aggregate_sc(x_hbm, src_hbm, dst_hbm, zeros_hbm, p_out,
                       sidx, didx, rows0, rows1, aggr_sh, gsem0, gsem1):
    cid = lax.axis_index("c")
    sid = lax.axis_index("s")
    wid = sid * NC + cid
    rows_per_tile = NP // NS
    pltpu.sync_copy(zeros_hbm.at[pl.ds(sid * rows_per_tile, rows_per_tile)],
                    aggr_sh.at[pl.ds(sid * rows_per_tile, rows_per_tile)])
    plsc.subcore_barrier()

    def stage(c, rows, gsem, issue_next):
        pltpu.make_async_copy(x_hbm.at[sidx.at[c]], rows, gsem).wait()
        pltpu.sync_copy(rows, aggr_sh.at[didx.at[c]], add=True)
        if issue_next:
            pltpu.async_copy(x_hbm.at[sidx.at[c + 2]], rows, gsem)

    def body(i, carry):
        c = 2 * i
        stage(c, rows0, gsem0, True)
        stage(c + 1, rows1, gsem1, True)
        return carry

    for g in range(E_NG):
        pltpu.sync_copy(src_hbm.at[wid * E_NG + g], sidx)
        pltpu.sync_copy(dst_hbm.at[wid * E_NG + g], didx)
        pltpu.async_copy(x_hbm.at[sidx.at[0]], rows0, gsem0)
        pltpu.async_copy(x_hbm.at[sidx.at[1]], rows1, gsem1)
        lax.fori_loop(0, (E_G - 2) // 2, body, 0)
        stage(E_G - 2, rows0, gsem0, False)
        stage(E_G - 1, rows1, gsem1, False)
    plsc.subcore_barrier()
    pltpu.sync_copy(aggr_sh.at[pl.ds(sid * rows_per_tile, rows_per_tile)],
                    p_out.at[pl.ds(cid * NP + sid * rows_per_tile, rows_per_tile)])


def _bn_layer(x, p, w_ref, b_ref, g_ref, be_ref, relu):
    y = (1.0 + SCALAR) * x + p
    z = jnp.dot(y, w_ref[:], preferred_element_type=jnp.float32) + b_ref[:]
    mean = jnp.mean(z, axis=0, keepdims=True)
    zc = z - mean
    var = jnp.mean(zc * zc, axis=0, keepdims=True)
    zn = zc * lax.rsqrt(var + BN_EPS) * g_ref[:] + be_ref[:]
    if relu:
        zn = jnp.maximum(zn, 0.0)
    return zn


def _layer_tc_body(x_ref, p_ref, w_ref, b_ref, g_ref, be_ref, o_ref, *, relu):
    o_ref[0:N, :] = _bn_layer(x_ref[0:N, :], p_ref[0:N, :] + p_ref[NP:NP + N, :],
                              w_ref, b_ref, g_ref, be_ref, relu)


def _layer_tc(x, p, w, b, g, be, relu):
    return pl.pallas_call(
        functools.partial(_layer_tc_body, relu=relu),
        out_shape=jax.ShapeDtypeStruct((NP, NHID), jnp.float32),
    )(x, p, w, b, g, be)


def _last_layer_pool_body(x_ref, p_ref, w_ref, b_ref, g_ref, be_ref,
                          batch_ref, pw_ref, pb_ref, o_ref):
    zn = _bn_layer(x_ref[0:N, :], p_ref[0:N, :] + p_ref[NP:NP + N, :],
                   w_ref, b_ref, g_ref, be_ref, relu=False)
    b = batch_ref[:]
    gids = lax.broadcasted_iota(jnp.int32, (NGRAPHS, N), 0)
    onehot = (gids == b).astype(jnp.float32)
    sums = jnp.dot(onehot, zn, preferred_element_type=jnp.float32)
    counts = jnp.maximum(jnp.sum(onehot, axis=1, keepdims=True), 1.0)
    pooled = sums / counts
    o_ref[:] = jnp.dot(pooled, pw_ref[:],
                       preferred_element_type=jnp.float32) + pb_ref[:]


def _last_layer_pool_tc(x, p, w, b, g, be, batch2, pw, pb):
    return pl.pallas_call(
        _last_layer_pool_body,
        out_shape=jax.ShapeDtypeStruct((NGRAPHS, NCLASS), jnp.float32),
    )(x, p, w, b, g, be, batch2, pw, pb)


def kernel(h, edge_index, pair_info, batch, atom_tables, conv_W, conv_b,
           bn_gamma, bn_beta, pred_W, pred_b):
    hp = jnp.pad(h, ((0, NP - N), (0, 0)))
    hoff = hp + (jnp.arange(NFEATCOLS, dtype=jnp.int32) * ATOM_VOCAB)[None, :]
    hoff = (hoff.T.reshape(NFEATCOLS, NW, A_NCH, A_CH)
            .transpose(1, 0, 2, 3).reshape(NW, NFEATCOLS * A_NCH, A_CH))
    tables_flat = jnp.pad(
        atom_tables.reshape(NFEATCOLS * ATOM_VOCAB, NHID),
        ((0, TBL_PAD - NFEATCOLS * ATOM_VOCAB), (0, 0)))
    pad_t = NP - N
    pad_src = jnp.broadcast_to(
        jnp.arange(pad_t, dtype=jnp.int32) * (N // pad_t), (NW, pad_t))
    pad_dst = jnp.broadcast_to(
        N + jnp.arange(pad_t, dtype=jnp.int32), (NW, pad_t))
    src = jnp.concatenate(
        [pair_info[0].reshape(NW, E // NW), pad_src], axis=1
    ).reshape(NW * E_NG, E_G, E_CH)
    dst = jnp.concatenate(
        [pair_info[1].reshape(NW, E // NW), pad_dst], axis=1
    ).reshape(NW * E_NG, E_G, E_CH)
    zeros = jnp.zeros((NP, NHID), jnp.float32)
    batch2 = batch.reshape(1, N)

    x = _atom_encode_sc(hoff, tables_flat)
    for layer in range(NLAYERS - 1):
        p = _edge_aggregate_sc(x, src, dst, zeros)
        x = _layer_tc(x, p, conv_W[layer], conv_b[layer].reshape(1, NHID),
                      bn_gamma[layer].reshape(1, NHID),
                      bn_beta[layer].reshape(1, NHID),
                      relu=True)
    p = _edge_aggregate_sc(x, src, dst, zeros)
    last = NLAYERS - 1
    return _last_layer_pool_tc(
        x, p, conv_W[last], conv_b[last].reshape(1, NHID),
        bn_gamma[last].reshape(1, NHID), bn_beta[last].reshape(1, NHID),
        batch2, pred_W, pred_b.reshape(1, NCLASS))
